# Initial kernel scaffold; baseline (speedup 1.0000x reference)
import jax
import jax.numpy as jnp
from jax.experimental import pallas as pl


def kernel(x, edge_index, W0, b0, W1, b1, Wp1, bp1, Wp2, bp2):
    raise NotImplementedError("write your pallas kernel here")


# speedup vs baseline: 7.9190x; 7.9190x over previous
import functools

import jax
import jax.numpy as jnp
from jax import lax
from jax.experimental import pallas as pl
from jax.experimental.pallas import tpu as pltpu
from jax.experimental.pallas import tpu_sc as plsc

NC = 2
NS = 16
CHUNK = 128


def _round_up(a, b):
    return (a + b - 1) // b * b


def _make_spmm(n_pad, e_pad):
    e_half = e_pad // NC
    per_tile = e_half // NS
    n_chunks = per_tile // CHUNK
    rows_per_tile = n_pad // NS
    wb_chunks = rows_per_tile // CHUNK

    out_type = [jax.ShapeDtypeStruct((NC * n_pad, 128), jnp.float32)]
    scratch = [
        pltpu.VMEM((CHUNK,), jnp.int32),
        pltpu.VMEM((CHUNK,), jnp.int32),
        pltpu.VMEM((CHUNK, 128), jnp.float32),
        pltpu.VMEM_SHARED((n_pad, 128), jnp.float32),
        pltpu.SemaphoreType.DMA,
    ]

    mesh = plsc.VectorSubcoreMesh(core_axis_name="c", subcore_axis_name="s")

    def body(z128_hbm, src_hbm, dst_hbm, y_hbm, acc_hbm,
             idx_s, idx_d, gbuf, acc_sh, sem):
        c = lax.axis_index("c")
        s = lax.axis_index("s")

        pltpu.sync_copy(z128_hbm, gbuf)

        r0 = s * rows_per_tile

        def zacc(k, _):
            pltpu.sync_copy(gbuf, acc_sh.at[pl.ds(r0 + k * CHUNK, CHUNK)])
            return 0

        lax.fori_loop(0, wb_chunks, zacc, 0)

        plsc.subcore_barrier()

        ebase = c * e_half + s * per_tile

        def acc_step(j, _):
            off = ebase + j * CHUNK
            pltpu.sync_copy(src_hbm.at[pl.ds(off, CHUNK)], idx_s)
            pltpu.sync_copy(dst_hbm.at[pl.ds(off, CHUNK)], idx_d)
            pltpu.async_copy(y_hbm.at[idx_d], gbuf, sem).wait()
            pltpu.sync_copy(gbuf, acc_sh.at[idx_s], add=True)
            return 0

        lax.fori_loop(0, n_chunks, acc_step, 0)

        plsc.subcore_barrier()

        def wback(k, _):
            r = r0 + k * CHUNK
            pltpu.sync_copy(acc_sh.at[pl.ds(r, CHUNK)], gbuf)
            pltpu.sync_copy(gbuf, acc_hbm.at[pl.ds(c * n_pad + r, CHUNK)])
            return 0

        lax.fori_loop(0, wb_chunks, wback, 0)

    return pl.kernel(body, out_type=out_type, mesh=mesh, scratch_types=scratch)


def _make_deg(n_pad, e_pad):
    e_half = e_pad // NC
    per_tile = e_half // NS
    n_chunks = per_tile // CHUNK
    rows_per_tile = n_pad // NS
    wb_chunks = rows_per_tile // CHUNK

    out_type = [jax.ShapeDtypeStruct((NC * n_pad, 128), jnp.float32)]
    scratch = [
        pltpu.VMEM((CHUNK,), jnp.int32),
        pltpu.VMEM((CHUNK, 128), jnp.float32),
        pltpu.VMEM_SHARED((n_pad, 128), jnp.float32),
    ]

    mesh = plsc.VectorSubcoreMesh(core_axis_name="c", subcore_axis_name="s")

    def body(z128_hbm, o128_hbm, src_hbm, deg_hbm, idx_s, obuf, deg_sh):
        c = lax.axis_index("c")
        s = lax.axis_index("s")
        r0 = s * rows_per_tile

        pltpu.sync_copy(z128_hbm, obuf)

        def zacc(k, _):
            pltpu.sync_copy(obuf, deg_sh.at[pl.ds(r0 + k * CHUNK, CHUNK)])
            return 0

        lax.fori_loop(0, wb_chunks, zacc, 0)

        pltpu.sync_copy(o128_hbm, obuf)

        plsc.subcore_barrier()

        ebase = c * e_half + s * per_tile

        def acc_step(j, _):
            pltpu.sync_copy(src_hbm.at[pl.ds(ebase + j * CHUNK, CHUNK)], idx_s)
            pltpu.sync_copy(obuf, deg_sh.at[idx_s], add=True)
            return 0

        lax.fori_loop(0, n_chunks, acc_step, 0)

        plsc.subcore_barrier()

        def wback(k, _):
            r = r0 + k * CHUNK
            pltpu.sync_copy(deg_sh.at[pl.ds(r, CHUNK)], obuf)
            pltpu.sync_copy(obuf, deg_hbm.at[pl.ds(c * n_pad + r, CHUNK)])
            return 0

        lax.fori_loop(0, wb_chunks, wback, 0)

    return pl.kernel(body, out_type=out_type, mesh=mesh, scratch_types=scratch)


def _tc_proj(x, w, n_pad):
    n = x.shape[0]
    h_dim = w.shape[1]

    def body(x_ref, w_ref, o_ref):
        o_ref[0:n, :] = jnp.dot(x_ref[...], w_ref[...],
                                preferred_element_type=jnp.float32)
        o_ref[n:n_pad, :] = jnp.zeros((n_pad - n, h_dim), jnp.float32)

    return pl.pallas_call(
        body, out_shape=jax.ShapeDtypeStruct((n_pad, h_dim), jnp.float32))(x, w)


def _tc_mid(acc, y, deg, b, w, n_pad):
    def body(acc_ref, y_ref, deg_ref, b_ref, w_ref, o_ref):
        inv = 1.0 / (deg_ref[0:n_pad, 0:1] + deg_ref[n_pad:2 * n_pad, 0:1] + 1.0)
        t = (acc_ref[0:n_pad] + acc_ref[n_pad:2 * n_pad] + y_ref[...]) * inv
        h = jnp.maximum(t + b_ref[...], 0.0)
        o_ref[...] = jnp.dot(h, w_ref[...], preferred_element_type=jnp.float32)

    return pl.pallas_call(
        body, out_shape=jax.ShapeDtypeStruct((n_pad, w.shape[1]), jnp.float32))(
            acc, y, deg, b, w)


def _tc_final(acc, y, deg, b, wp1, bp1, wp2, bp2, n, n_pad):
    def body(acc_ref, y_ref, deg_ref, b_ref, wp1_ref, bp1_ref, wp2_ref,
             bp2_ref, o_ref):
        inv = 1.0 / (deg_ref[0:n, 0:1] + deg_ref[n_pad:n_pad + n, 0:1] + 1.0)
        t = (acc_ref[0:n] + acc_ref[n_pad:n_pad + n] + y_ref[0:n]) * inv
        h = jnp.maximum(t + b_ref[...], 0.0)
        t2 = jnp.dot(h, wp1_ref[...], preferred_element_type=jnp.float32) + bp1_ref[...]
        o_ref[...] = jnp.dot(t2, wp2_ref[...],
                             preferred_element_type=jnp.float32) + bp2_ref[...]

    return pl.pallas_call(
        body, out_shape=jax.ShapeDtypeStruct((n, wp2.shape[1]), jnp.float32))(
            acc, y, deg, b, wp1, bp1, wp2, bp2)


def kernel(x, edge_index, W0, b0, W1, b1, Wp1, bp1, Wp2, bp2):
    n = x.shape[0]
    e = edge_index.shape[1]
    n_pad = _round_up(n + 1, NS * CHUNK)
    e_pad = _round_up(e, NC * NS * CHUNK)

    pad = e_pad - e
    src_p = jnp.concatenate([edge_index[0], jnp.full((pad,), n, jnp.int32)])
    dst_p = jnp.concatenate([edge_index[1], jnp.zeros((pad,), jnp.int32)])

    spmm = _make_spmm(n_pad, e_pad)
    deg_hist = _make_deg(n_pad, e_pad)

    b0_2 = b0.reshape(1, -1)
    b1_2 = b1.reshape(1, -1)
    bp1_2 = bp1.reshape(1, -1)
    bp2_2 = bp2.reshape(1, -1)

    z128 = jnp.zeros((CHUNK, 128), jnp.float32)
    o128 = jnp.ones((CHUNK, 128), jnp.float32)

    (deg,) = deg_hist(z128, o128, src_p)
    y0 = _tc_proj(x, W0, n_pad)
    (acc1,) = spmm(z128, src_p, dst_p, y0)
    y1 = _tc_mid(acc1, y0, deg, b0_2, W1, n_pad)
    (acc2,) = spmm(z128, src_p, dst_p, y1)
    out = _tc_final(acc2, y1, deg, b1_2, Wp1, bp1_2, Wp2, bp2_2, n, n_pad)
    return out
